# baseline (speedup 1.0000x reference)
import functools

import jax
import jax.numpy as jnp
from jax import lax
from jax.experimental import pallas as pl
from jax.experimental.pallas import tpu as pltpu
from jax.experimental.pallas import tpu_sc as plsc

NUM_CLASS = 100000
CTX_DIM = 512
N_CLS_CTX = 4
N_PRE = 5
N_SUF = 68
CLIP_LEN = 77
BATCH = 4096

ROW = CLIP_LEN * CTX_DIM
CLS_W = N_CLS_CTX * CTX_DIM
PRE_W = N_PRE * CTX_DIM
SUF_OFF = PRE_W + CLS_W
SUF_W = N_SUF * CTX_DIM

NC = 2
NS = 16
NW = NC * NS
BPW = BATCH // NW
K = 16


@functools.partial(
    pl.kernel,
    mesh=plsc.VectorSubcoreMesh(core_axis_name="c", subcore_axis_name="s"),
    out_type=jax.ShapeDtypeStruct((BATCH, ROW), jnp.float32),
    scratch_types=[
        pltpu.VMEM((BPW,), jnp.int32),
        pltpu.VMEM((K, CLS_W), jnp.float32),
        pltpu.VMEM((1, ROW), jnp.float32),
        pltpu.VMEM((1, ROW), jnp.float32),
        pltpu.SemaphoreType.DMA,
        pltpu.SemaphoreType.DMA,
        pltpu.SemaphoreType.DMA,
    ],
    compiler_params=pltpu.CompilerParams(use_tc_tiling_on_sc=False),
)
def _prompt_assemble(label_h, cls_h, pre_h, suf_h, out_h,
                     idx_v, cls_v, blk0, blk1, gsem, osem0, osem1):
    cid = lax.axis_index("c")
    sid = lax.axis_index("s")
    wid = sid * NC + cid
    base = wid * BPW

    pltpu.sync_copy(label_h.at[pl.ds(base, BPW)], idx_v)
    for blk in (blk0, blk1):
        pltpu.sync_copy(pre_h, blk.at[:, pl.ds(0, PRE_W)])
        pltpu.sync_copy(suf_h, blk.at[:, pl.ds(SUF_OFF, SUF_W)])

    def chunk(c, _):
        pltpu.async_copy(cls_h.at[idx_v.at[pl.ds(c * K, K)]], cls_v, gsem).wait()

        def pair(p, _):
            for b, (blk, osem) in enumerate(((blk0, osem0), (blk1, osem1))):
                j = p * 2 + b
                g = base + c * K + j

                @pl.when(c * K + j >= 2)
                def _wait_prev():
                    pltpu.make_async_copy(blk, out_h.at[pl.ds(g - 2, 1)], osem).wait()

                for t in range(CLS_W // 16):
                    blk[0, pl.ds(PRE_W + t * 16, 16)] = cls_v[j, pl.ds(t * 16, 16)]
                pltpu.async_copy(blk, out_h.at[pl.ds(g, 1)], osem)
            return None

        lax.fori_loop(0, K // 2, pair, None)
        return None

    lax.fori_loop(0, BPW // K, chunk, None)
    pltpu.make_async_copy(blk0, out_h.at[pl.ds(base + BPW - 2, 1)], osem0).wait()
    pltpu.make_async_copy(blk1, out_h.at[pl.ds(base + BPW - 1, 1)], osem1).wait()


def kernel(label, cls_ctx, token_prefix, token_suffix):
    out = _prompt_assemble(
        label.astype(jnp.int32),
        cls_ctx.reshape(NUM_CLASS, CLS_W),
        token_prefix.reshape(1, PRE_W),
        token_suffix.reshape(1, SUF_W),
    )
    return out.reshape(BATCH, CLIP_LEN, CTX_DIM)


# speedup vs baseline: 3.3284x; 3.3284x over previous
import functools

import jax
import jax.numpy as jnp
from jax import lax
from jax.experimental import pallas as pl
from jax.experimental.pallas import tpu as pltpu
from jax.experimental.pallas import tpu_sc as plsc

NUM_CLASS = 100000
CTX_DIM = 512
N_CLS_CTX = 4
N_PRE = 5
N_SUF = 68
CLIP_LEN = 77
BATCH = 4096
LANES = 16
VPR = CTX_DIM // LANES

NC = 2
NS = 16
NW = NC * NS
BPW = BATCH // NW
K = 8


def _copy_row(src, srow, dst, drow):
    for v in range(VPR):
        dst[0, drow, pl.ds(v * LANES, LANES)] = src[0, srow, pl.ds(v * LANES, LANES)]


@functools.partial(
    pl.kernel,
    mesh=plsc.VectorSubcoreMesh(core_axis_name="c", subcore_axis_name="s"),
    out_type=jax.ShapeDtypeStruct((BATCH, CLIP_LEN, CTX_DIM), jnp.float32),
    scratch_types=[
        pltpu.VMEM((BPW,), jnp.int32),
        pltpu.VMEM((K, N_CLS_CTX, CTX_DIM), jnp.float32),
        pltpu.VMEM((1, 8, CTX_DIM), jnp.float32),
        pltpu.VMEM((1, CLIP_LEN, CTX_DIM), jnp.float32),
        pltpu.VMEM((1, CLIP_LEN, CTX_DIM), jnp.float32),
        pltpu.SemaphoreType.DMA,
        pltpu.SemaphoreType.DMA,
        pltpu.SemaphoreType.DMA,
    ],
)
def _prompt_assemble(label_h, cls_h, pre_h, suf_h, out_h,
                     idx_v, cls_v, stage_v, blk0, blk1, gsem, osem0, osem1):
    cid = lax.axis_index("c")
    sid = lax.axis_index("s")
    wid = sid * NC + cid
    base = wid * BPW

    pltpu.sync_copy(label_h.at[pl.ds(base, BPW)], idx_v)

    pltpu.sync_copy(pre_h, stage_v.at[:, pl.ds(0, N_PRE)])
    for r in range(N_PRE):
        _copy_row(stage_v, r, blk0, r)
    for c in range(N_SUF // 8):
        pltpu.sync_copy(suf_h.at[:, pl.ds(c * 8, 8)], stage_v)
        for r in range(8):
            _copy_row(stage_v, r, blk0, N_PRE + N_CLS_CTX + c * 8 + r)
    pltpu.sync_copy(suf_h.at[:, pl.ds(N_SUF - 4, 4)], stage_v.at[:, pl.ds(0, 4)])
    for r in range(4):
        _copy_row(stage_v, r, blk0, CLIP_LEN - 4 + r)
    for r in range(CLIP_LEN):
        _copy_row(blk0, r, blk1, r)

    def chunk(c, _):
        pltpu.async_copy(cls_h.at[idx_v.at[pl.ds(c * K, K)]], cls_v, gsem).wait()

        def pair(p, _):
            for b, (blk, osem) in enumerate(((blk0, osem0), (blk1, osem1))):
                j = p * 2 + b
                g = base + c * K + j

                @pl.when(c * K + j >= 2)
                def _wait_prev():
                    pltpu.make_async_copy(blk, out_h.at[pl.ds(g - 2, 1)], osem).wait()

                for r in range(N_CLS_CTX):
                    for v in range(VPR):
                        blk[0, N_PRE + r, pl.ds(v * LANES, LANES)] = (
                            cls_v[j, r, pl.ds(v * LANES, LANES)])
                pltpu.async_copy(blk, out_h.at[pl.ds(g, 1)], osem)
            return None

        lax.fori_loop(0, K // 2, pair, None)
        return None

    lax.fori_loop(0, BPW // K, chunk, None)
    pltpu.make_async_copy(blk0, out_h.at[pl.ds(base + BPW - 2, 1)], osem0).wait()
    pltpu.make_async_copy(blk1, out_h.at[pl.ds(base + BPW - 1, 1)], osem1).wait()


def kernel(label, cls_ctx, token_prefix, token_suffix):
    return _prompt_assemble(label.astype(jnp.int32), cls_ctx, token_prefix, token_suffix)


# speedup vs baseline: 7.6478x; 2.2978x over previous
import functools

import jax
import jax.numpy as jnp
from jax import lax
from jax.experimental import pallas as pl
from jax.experimental.pallas import tpu as pltpu
from jax.experimental.pallas import tpu_sc as plsc

NUM_CLASS = 100000
CTX_DIM = 512
N_CLS_CTX = 4
N_PRE = 5
N_SUF = 68
N_BCAST = N_PRE + N_SUF
CLIP_LEN = 77
BATCH = 4096
LANES = 16
VPR = CTX_DIM // LANES

NC = 2
NS = 16
NW = NC * NS
BPW = BATCH // NW
K = 8
SLICE = 128
NSLICE = BATCH // SLICE
UPW = (N_BCAST * NSLICE) // NW


@functools.partial(
    pl.kernel,
    mesh=plsc.VectorSubcoreMesh(core_axis_name="c", subcore_axis_name="s"),
    out_type=jax.ShapeDtypeStruct((CLIP_LEN, BATCH, CTX_DIM), jnp.float32),
    scratch_types=[
        pltpu.VMEM((BPW,), jnp.int32),
        pltpu.VMEM((K, N_CLS_CTX, CTX_DIM), jnp.float32),
        pltpu.VMEM((N_CLS_CTX, K, CTX_DIM), jnp.float32),
        pltpu.VMEM((1, SLICE, CTX_DIM), jnp.float32),
        pltpu.VMEM((8, CTX_DIM), jnp.float32),
        pltpu.SemaphoreType.DMA,
        pltpu.SemaphoreType.DMA,
        pltpu.SemaphoreType.DMA,
    ],
)
def _prompt_assemble(label_h, cls_h, ps_h, out_h,
                     idx_v, cls_v, clsT_v, rep_v, stage_v, gsem, csem, bsem):
    cid = lax.axis_index("c")
    sid = lax.axis_index("s")
    wid = sid * NC + cid
    base = wid * BPW

    pltpu.sync_copy(label_h.at[pl.ds(base, BPW)], idx_v)

    def chunk(c, _):
        pltpu.async_copy(cls_h.at[idx_v.at[pl.ds(c * K, K)]], cls_v, gsem).wait()

        @pl.when(c > 0)
        def _drain_prev():
            for r in range(N_CLS_CTX):
                pltpu.make_async_copy(
                    clsT_v.at[pl.ds(r, 1)],
                    out_h.at[pl.ds(N_PRE + r, 1), pl.ds(base, K)], csem).wait()

        def tr(j, _):
            for r in range(N_CLS_CTX):
                for v in range(VPR):
                    clsT_v[r, j, pl.ds(v * LANES, LANES)] = (
                        cls_v[j, r, pl.ds(v * LANES, LANES)])
            return None

        lax.fori_loop(0, K, tr, None)
        for r in range(N_CLS_CTX):
            pltpu.async_copy(
                clsT_v.at[pl.ds(r, 1)],
                out_h.at[pl.ds(N_PRE + r, 1), pl.ds(base + c * K, K)], csem)
        return None

    lax.fori_loop(0, BPW // K, chunk, None)
    for r in range(N_CLS_CTX):
        pltpu.make_async_copy(
            clsT_v.at[pl.ds(r, 1)],
            out_h.at[pl.ds(N_PRE + r, 1), pl.ds(base, K)], csem).wait()

    u0 = wid * UPW
    tb_lo = u0 // NSLICE
    tb_hi = (u0 + UPW - 1) // NSLICE

    def per_tb(tb, _):
        pltpu.sync_copy(ps_h.at[pl.ds((tb // 8) * 8, 8)], stage_v)
        tbm = tb % 8
        rows = [stage_v[tbm, pl.ds(v * LANES, LANES)] for v in range(VPR)]

        def fill(rw, _):
            for v in range(VPR):
                rep_v[0, rw, pl.ds(v * LANES, LANES)] = rows[v]
            return None

        lax.fori_loop(0, SLICE, fill, None)

        t = jnp.where(tb < N_PRE, tb, tb + N_CLS_CTX)
        klo = jnp.maximum(u0 - tb * NSLICE, 0)
        khi = jnp.minimum(u0 + UPW - tb * NSLICE, NSLICE)

        def issue(k, _):
            pltpu.async_copy(rep_v, out_h.at[pl.ds(t, 1), pl.ds(k * SLICE, SLICE)], bsem)
            return None

        lax.fori_loop(klo, khi, issue, None)

        def drain(k, _):
            pltpu.make_async_copy(
                rep_v, out_h.at[pl.ds(t, 1), pl.ds(0, SLICE)], bsem).wait()
            return None

        lax.fori_loop(0, khi - klo, drain, None)
        return None

    lax.fori_loop(tb_lo, tb_hi + 1, per_tb, None)


def kernel(label, cls_ctx, token_prefix, token_suffix):
    ps = jnp.concatenate([token_prefix, token_suffix], axis=1)
    ps = jnp.pad(ps, ((0, 0), (0, 80 - N_BCAST), (0, 0))).reshape(80, CTX_DIM)
    out = _prompt_assemble(label.astype(jnp.int32), cls_ctx, ps)
    return out.transpose(1, 0, 2)


# speedup vs baseline: 7.6744x; 1.0035x over previous
import functools

import jax
import jax.numpy as jnp
from jax import lax
from jax.experimental import pallas as pl
from jax.experimental.pallas import tpu as pltpu
from jax.experimental.pallas import tpu_sc as plsc

NUM_CLASS = 100000
CTX_DIM = 512
N_CLS_CTX = 4
N_PRE = 5
N_SUF = 68
N_BCAST = N_PRE + N_SUF
CLIP_LEN = 77
BATCH = 4096
LANES = 16
VPR = CTX_DIM // LANES

NC = 2
NS = 16
NW = NC * NS
BPW = BATCH // NW
K = 8
SLICE = 128
NSLICE = BATCH // SLICE
UPW = (N_BCAST * NSLICE) // NW


@functools.partial(
    pl.kernel,
    mesh=plsc.VectorSubcoreMesh(core_axis_name="c", subcore_axis_name="s"),
    out_type=jax.ShapeDtypeStruct((CLIP_LEN, BATCH, CTX_DIM), jnp.float32),
    scratch_types=[
        pltpu.VMEM((BPW,), jnp.int32),
        pltpu.VMEM((K, N_CLS_CTX, CTX_DIM), jnp.float32),
        pltpu.VMEM((N_CLS_CTX, K, CTX_DIM), jnp.float32),
        pltpu.VMEM((1, SLICE, CTX_DIM), jnp.float32),
        pltpu.VMEM((8, CTX_DIM), jnp.float32),
        pltpu.SemaphoreType.DMA,
        pltpu.SemaphoreType.DMA,
        pltpu.SemaphoreType.DMA,
    ],
)
def _prompt_assemble(label_h, cls_h, ps_h, out_h,
                     idx_v, cls_v, clsT_v, rep_v, stage_v, gsem, csem, bsem):
    cid = lax.axis_index("c")
    sid = lax.axis_index("s")
    wid = sid * NC + cid
    base = wid * BPW

    u0 = wid * UPW
    tb_lo = u0 // NSLICE
    tb_hi = (u0 + UPW - 1) // NSLICE

    def k_bounds(tb):
        klo = jnp.maximum(u0 - tb * NSLICE, 0)
        khi = jnp.minimum(u0 + UPW - tb * NSLICE, NSLICE)
        return klo, khi

    def fill_rep(tb):
        pltpu.sync_copy(ps_h.at[pl.ds((tb // 8) * 8, 8)], stage_v)
        tbm = tb % 8
        rows = [stage_v[tbm, pl.ds(v * LANES, LANES)] for v in range(VPR)]

        def fill(rw, _):
            for v in range(VPR):
                rep_v[0, rw, pl.ds(v * LANES, LANES)] = rows[v]
            return None

        lax.fori_loop(0, SLICE, fill, None)

    def issue_tb(tb):
        t = jnp.where(tb < N_PRE, tb, tb + N_CLS_CTX)
        klo, khi = k_bounds(tb)

        def issue(k, _):
            pltpu.async_copy(rep_v, out_h.at[pl.ds(t, 1), pl.ds(k * SLICE, SLICE)], bsem)
            return None

        lax.fori_loop(klo, khi, issue, None)

    def drain_tb(tb):
        klo, khi = k_bounds(tb)

        def drain(k, _):
            pltpu.make_async_copy(
                rep_v, out_h.at[pl.ds(0, 1), pl.ds(0, SLICE)], bsem).wait()
            return None

        lax.fori_loop(0, khi - klo, drain, None)

    pltpu.sync_copy(label_h.at[pl.ds(base, BPW)], idx_v)

    fill_rep(tb_lo)
    issue_tb(tb_lo)

    def chunk(c, _):
        pltpu.async_copy(cls_h.at[idx_v.at[pl.ds(c * K, K)]], cls_v, gsem).wait()

        @pl.when(c > 0)
        def _drain_prev():
            for r in range(N_CLS_CTX):
                pltpu.make_async_copy(
                    clsT_v.at[pl.ds(r, 1)],
                    out_h.at[pl.ds(N_PRE + r, 1), pl.ds(base, K)], csem).wait()

        def tr(j, _):
            for r in range(N_CLS_CTX):
                for v in range(VPR):
                    clsT_v[r, j, pl.ds(v * LANES, LANES)] = (
                        cls_v[j, r, pl.ds(v * LANES, LANES)])
            return None

        lax.fori_loop(0, K, tr, None)
        for r in range(N_CLS_CTX):
            pltpu.async_copy(
                clsT_v.at[pl.ds(r, 1)],
                out_h.at[pl.ds(N_PRE + r, 1), pl.ds(base + c * K, K)], csem)
        return None

    lax.fori_loop(0, BPW // K, chunk, None)

    def per_tb(tb, _):
        drain_tb(tb - 1)
        fill_rep(tb)
        issue_tb(tb)
        return None

    lax.fori_loop(tb_lo + 1, tb_hi + 1, per_tb, None)

    drain_tb(tb_hi)
    for r in range(N_CLS_CTX):
        pltpu.make_async_copy(
            clsT_v.at[pl.ds(r, 1)],
            out_h.at[pl.ds(N_PRE + r, 1), pl.ds(base, K)], csem).wait()


def kernel(label, cls_ctx, token_prefix, token_suffix):
    ps = jnp.concatenate([token_prefix, token_suffix], axis=1)
    ps = jnp.pad(ps, ((0, 0), (0, 80 - N_BCAST), (0, 0))).reshape(80, CTX_DIM)
    out = _prompt_assemble(label.astype(jnp.int32), cls_ctx, ps)
    return out.transpose(1, 0, 2)
